# baseline (speedup 1.0000x reference)
import jax
import jax.numpy as jnp
from jax import lax
from jax.experimental import pallas as pl
from jax.experimental.pallas import tpu as pltpu
from jax.experimental.pallas import tpu_sc as plsc

_N = 8192
_K = 819
_CH = 512
_NV = _N // 16


def _iota16():
    return lax.broadcasted_iota(jnp.int32, (16,), 0)


def _key(s):
    return s ^ ((s >> 31) & jnp.int32(0x7FFFFFFF))


def _suffix_sum(v):
    r = lax.rev(v, (0,))
    return lax.rev(plsc.cumsum(r), (0,))


def _scan_level(hist, nv, k_cur, iota):
    sums = [jnp.sum(hist[pl.ds(j * 16, 16)]) for j in range(nv)]
    bsel = jnp.int32(0)
    asel = jnp.int32(0)
    above = jnp.int32(0)
    for j in reversed(range(nv)):
        found = jnp.logical_and(above < k_cur, above + sums[j] >= k_cur)
        bsel = jnp.where(found, jnp.int32(j), bsel)
        asel = jnp.where(found, above, asel)
        above = above + sums[j]
    v = hist[pl.ds(bsel * 16, 16)]
    sv = asel + _suffix_sum(v)
    al = sv - v
    ml = jnp.logical_and(sv >= k_cur, al < k_cur)
    lane = jnp.sum(jnp.where(ml, iota, 0))
    above_b = jnp.sum(jnp.where(ml, al, 0))
    return bsel * 16 + lane, above_b


def _sc_body(bits_hbm, out_hbm, bitsv, hist, outb, cand, candi):
    sid = lax.axis_index("s")
    iota = _iota16()
    kk = jnp.int32(_K)
    one16 = jnp.full((16,), 1, jnp.int32)
    zero16 = jnp.full((16,), 0, jnp.int32)

    pltpu.sync_copy(bits_hbm, bitsv)

    for j in range(32):
        hist[pl.ds(j * 16, 16)] = zero16

    def _l1(i, c):
        for u in range(8):
            kv = _key(bitsv[pl.ds((i * 8 + u) * 16, 16)])
            plsc.addupdate_scatter(hist, [(kv >> 23) + 256], one16)
        return c

    lax.fori_loop(0, 64, _l1, jnp.int32(0))
    b1, above = _scan_level(hist, 32, kk, iota)
    k_cur = kk - above

    def _cp(i, off):
        for u in range(8):
            g = (i * 8 + u) * 16
            kv = _key(bitsv[pl.ds(g, 16)])
            part = ((kv >> 23) + 256) == b1
            plsc.store_compressed(cand.at[pl.ds(off, 16)], kv, mask=part)
            plsc.store_compressed(candi.at[pl.ds(off, 16)], g + iota, mask=part)
            off = off + jnp.sum(jnp.where(part, 1, 0))
        return off

    nc = lax.fori_loop(0, 64, _cp, jnp.int32(0))
    nloop = (nc + 15) >> 4

    top18 = jnp.int32(0)
    top27 = jnp.int32(0)
    tstar = jnp.int32(0)
    for li in range(1, 4):
        nv = 32 if li < 3 else 2
        for j in range(nv):
            hist[pl.ds(j * 16, 16)] = zero16

        def _acc(i, c, li=li, top18=top18, top27=top27):
            kv = cand[pl.ds(i * 16, 16)]
            valid = (i * 16 + iota) < nc
            if li == 1:
                b = lax.shift_right_logical(kv, 14) & jnp.int32(0x1FF)
            elif li == 2:
                b = lax.shift_right_logical(kv, 5) & jnp.int32(0x1FF)
                valid = jnp.logical_and(
                    valid, lax.shift_right_logical(kv, 14) == top18)
            else:
                b = kv & jnp.int32(0x1F)
                valid = jnp.logical_and(
                    valid, lax.shift_right_logical(kv, 5) == top27)
            plsc.addupdate_scatter(hist, [b], one16, mask=valid)
            return c

        lax.fori_loop(0, nloop, _acc, jnp.int32(0))
        bb, above = _scan_level(hist, nv, k_cur, iota)
        k_cur = k_cur - above
        if li == 1:
            top18 = ((b1 ^ jnp.int32(256)) << 9) | bb
        elif li == 2:
            top27 = (top18 << 9) | bb
        else:
            tstar = (top27 << 5) | bb

    base = sid * _CH

    def _pre(i, c):
        kv = cand[pl.ds(i * 16, 16)]
        gi = candi[pl.ds(i * 16, 16)]
        sel = jnp.logical_and((i * 16 + iota) < nc, kv == tstar)
        sel = jnp.logical_and(sel, gi < base)
        return c + jnp.sum(jnp.where(sel, 1, 0))

    carry = lax.fori_loop(0, nloop, _pre, jnp.int32(0))

    for j in range(32):
        kv = _key(bitsv[pl.ds(base + j * 16, 16)])
        eq = kv == tstar
        e = jnp.where(eq, 1, 0)
        ci = plsc.cumsum(e)
        sel = jnp.logical_and(eq, (carry + ci) <= k_cur)
        hit = jnp.logical_or(kv > tstar, sel)
        outb[pl.ds(j * 16, 16)] = jnp.where(hit, jnp.float32(1.0),
                                            jnp.float32(0.0))
        carry = carry + jnp.sum(jnp.where(iota == 15, ci, 0))
    pltpu.sync_copy(outb, out_hbm.at[pl.ds(base, _CH)])


@jax.jit
def _sc_topk_mask(gate_bits):
    mesh = plsc.VectorSubcoreMesh(core_axis_name="c", subcore_axis_name="s",
                                  num_cores=1, num_subcores=16)
    f = pl.kernel(
        _sc_body,
        out_type=jax.ShapeDtypeStruct((_N,), jnp.float32),
        mesh=mesh,
        compiler_params=pltpu.CompilerParams(needs_layout_passes=False),
        scratch_types=[
            pltpu.VMEM((_N,), jnp.int32),
            pltpu.VMEM((512,), jnp.int32),
            pltpu.VMEM((_CH,), jnp.float32),
            pltpu.VMEM((_N,), jnp.int32),
            pltpu.VMEM((_N,), jnp.int32),
        ],
    )
    return f(gate_bits)


def kernel(x, gate_scores):
    bits = lax.bitcast_convert_type(gate_scores, jnp.int32)
    return _sc_topk_mask(bits).astype(x.dtype)


# speedup vs baseline: 1.0341x; 1.0341x over previous
import jax
import jax.numpy as jnp
from jax import lax
from jax.experimental import pallas as pl
from jax.experimental.pallas import tpu as pltpu
from jax.experimental.pallas import tpu_sc as plsc

_N = 8192
_K = 819
_CH = 512
_NV = _N // 16


def _iota16():
    return lax.broadcasted_iota(jnp.int32, (16,), 0)


def _key(s):
    return s ^ ((s >> 31) & jnp.int32(0x7FFFFFFF))


def _lane(v, i):
    return lax.squeeze(lax.slice(v, (i,), (i + 1,)), dimensions=(0,))


def _suffix_sum(v):
    r = lax.rev(v, (0,))
    return lax.rev(plsc.cumsum(r), (0,))


def _scan_level(hist, nv, k_cur, iota):
    sums = [jnp.sum(hist[pl.ds(j * 16, 16)]) for j in range(nv)]
    bsel = jnp.int32(0)
    asel = jnp.int32(0)
    above = jnp.int32(0)
    for j in reversed(range(nv)):
        found = jnp.logical_and(above < k_cur, above + sums[j] >= k_cur)
        bsel = jnp.where(found, jnp.int32(j), bsel)
        asel = jnp.where(found, above, asel)
        above = above + sums[j]
    v = hist[pl.ds(bsel * 16, 16)]
    sv = asel + _suffix_sum(v)
    al = sv - v
    ml = jnp.logical_and(sv >= k_cur, al < k_cur)
    lane = jnp.sum(jnp.where(ml, iota, 0))
    above_b = jnp.sum(jnp.where(ml, al, 0))
    return bsel * 16 + lane, above_b


def _sc_body(bits_hbm, out_hbm, bitsv, hist, outb, cand, candi):
    sid = lax.axis_index("s")
    iota = _iota16()
    kk = jnp.int32(_K)
    one16 = jnp.full((16,), 1, jnp.int32)
    zero16 = jnp.full((16,), 0, jnp.int32)

    pltpu.sync_copy(bits_hbm, bitsv)

    for j in range(32):
        hist[pl.ds(j * 16, 16)] = zero16

    def _l1(i, c):
        for u in range(8):
            kv = _key(bitsv[pl.ds((i * 8 + u) * 16, 16)])
            plsc.addupdate_scatter(hist, [(kv >> 23) + 256], one16)
        return c

    lax.fori_loop(0, 64, _l1, jnp.int32(0))
    b1, above = _scan_level(hist, 32, kk, iota)
    k_cur = kk - above

    def _cp(i, off):
        for u in range(8):
            g = (i * 8 + u) * 16
            kv = _key(bitsv[pl.ds(g, 16)])
            part = ((kv >> 23) + 256) == b1
            plsc.store_compressed(cand.at[pl.ds(off, 16)], kv, mask=part)
            plsc.store_compressed(candi.at[pl.ds(off, 16)], g + iota, mask=part)
            off = off + _lane(plsc.all_reduce_population_count(part), 0)
        return off

    nc = lax.fori_loop(0, 64, _cp, jnp.int32(0))
    nloop = (nc + 15) >> 4

    top18 = jnp.int32(0)
    top27 = jnp.int32(0)
    tstar = jnp.int32(0)
    for li in range(1, 4):
        nv = 32 if li < 3 else 2
        for j in range(nv):
            hist[pl.ds(j * 16, 16)] = zero16

        def _acc(i, c, li=li, top18=top18, top27=top27):
            kv = cand[pl.ds(i * 16, 16)]
            valid = (i * 16 + iota) < nc
            if li == 1:
                b = lax.shift_right_logical(kv, 14) & jnp.int32(0x1FF)
            elif li == 2:
                b = lax.shift_right_logical(kv, 5) & jnp.int32(0x1FF)
                valid = jnp.logical_and(
                    valid, lax.shift_right_logical(kv, 14) == top18)
            else:
                b = kv & jnp.int32(0x1F)
                valid = jnp.logical_and(
                    valid, lax.shift_right_logical(kv, 5) == top27)
            plsc.addupdate_scatter(hist, [b], one16, mask=valid)
            return c

        lax.fori_loop(0, nloop, _acc, jnp.int32(0))
        bb, above = _scan_level(hist, nv, k_cur, iota)
        k_cur = k_cur - above
        if li == 1:
            top18 = ((b1 ^ jnp.int32(256)) << 9) | bb
        elif li == 2:
            top27 = (top18 << 9) | bb
        else:
            tstar = (top27 << 5) | bb

    base = sid * _CH

    def _pre(i, c):
        kv = cand[pl.ds(i * 16, 16)]
        gi = candi[pl.ds(i * 16, 16)]
        sel = jnp.logical_and((i * 16 + iota) < nc, kv == tstar)
        sel = jnp.logical_and(sel, gi < base)
        return c + _lane(plsc.all_reduce_population_count(sel), 0)

    carry = lax.fori_loop(0, nloop, _pre, jnp.int32(0))

    for j in range(32):
        kv = _key(bitsv[pl.ds(base + j * 16, 16)])
        eq = kv == tstar
        e = jnp.where(eq, 1, 0)
        ci = plsc.cumsum(e)
        sel = jnp.logical_and(eq, (carry + ci) <= k_cur)
        hit = jnp.logical_or(kv > tstar, sel)
        outb[pl.ds(j * 16, 16)] = jnp.where(hit, jnp.float32(1.0),
                                            jnp.float32(0.0))
        carry = carry + _lane(ci, 15)
    pltpu.sync_copy(outb, out_hbm.at[pl.ds(base, _CH)])


@jax.jit
def _sc_topk_mask(gate_bits):
    mesh = plsc.VectorSubcoreMesh(core_axis_name="c", subcore_axis_name="s",
                                  num_cores=1, num_subcores=16)
    f = pl.kernel(
        _sc_body,
        out_type=jax.ShapeDtypeStruct((_N,), jnp.float32),
        mesh=mesh,
        compiler_params=pltpu.CompilerParams(needs_layout_passes=False),
        scratch_types=[
            pltpu.VMEM((_N,), jnp.int32),
            pltpu.VMEM((512,), jnp.int32),
            pltpu.VMEM((_CH,), jnp.float32),
            pltpu.VMEM((_N,), jnp.int32),
            pltpu.VMEM((_N,), jnp.int32),
        ],
    )
    return f(gate_bits)


def kernel(x, gate_scores):
    bits = lax.bitcast_convert_type(gate_scores, jnp.int32)
    return _sc_topk_mask(bits).astype(x.dtype)


# speedup vs baseline: 11.4859x; 11.1069x over previous
import jax
import jax.numpy as jnp
from jax.experimental import pallas as pl

_N = 8192
_K = 819
_R = 64
_C = 128


def _body(g_ref, o_ref):
    g = g_ref[...]
    u = jax.lax.bitcast_convert_type(g, jnp.uint32)
    sign = u >> jnp.uint32(31)
    flip = jnp.uint32(0x80000000) + sign * jnp.uint32(0x7FFFFFFF)
    key = u ^ flip

    kk = jnp.int32(_K)

    tstar = jnp.uint32(0)
    rounds = [(29 - 3 * r, 7) for r in range(10)] + [(0, 3)]
    for shift, nc in rounds:
        cnts = [
            jnp.sum((key >= (tstar + jnp.uint32(i << shift)))
                    .astype(jnp.int32))
            for i in range(1, nc + 1)
        ]
        m = jnp.uint32(0)
        for c in cnts:
            m = m + (c >= kk).astype(jnp.uint32)
        tstar = tstar + (m << jnp.uint32(shift))

    gt = key > tstar
    eq = key == tstar
    c = jnp.sum(gt.astype(jnp.int32))
    need = (kk - c).astype(jnp.float32)

    e = eq.astype(jnp.float32)
    i1 = jax.lax.broadcasted_iota(jnp.int32, (_C, _C), 0)
    j1 = jax.lax.broadcasted_iota(jnp.int32, (_C, _C), 1)
    upper = (i1 <= j1).astype(jnp.float32)
    rowcum = jnp.dot(e, upper, preferred_element_type=jnp.float32)

    i2 = jax.lax.broadcasted_iota(jnp.int32, (_R, _R), 0)
    j2 = jax.lax.broadcasted_iota(jnp.int32, (_R, _R), 1)
    strict_lower = (j2 < i2).astype(jnp.float32)
    colpref = jnp.dot(strict_lower, e, preferred_element_type=jnp.float32)
    row_prefix = jnp.sum(colpref, axis=1, keepdims=True)

    rank = rowcum + row_prefix
    sel = jnp.logical_and(eq, rank <= need)
    o_ref[...] = jnp.logical_or(gt, sel).astype(jnp.float32)


def kernel(x, gate_scores):
    g2 = gate_scores.reshape(_R, _C)
    mask = pl.pallas_call(
        _body,
        out_shape=jax.ShapeDtypeStruct((_R, _C), jnp.float32),
    )(g2)
    return mask.reshape(_N).astype(x.dtype)


# speedup vs baseline: 12.0393x; 1.0482x over previous
import jax
import jax.numpy as jnp
from jax.experimental import pallas as pl

_N = 8192
_K = 819
_R = 64
_C = 128


def _body(g_ref, o_ref):
    g = g_ref[...]
    u = jax.lax.bitcast_convert_type(g, jnp.uint32)
    sign = u >> jnp.uint32(31)
    flip = jnp.uint32(0x80000000) + sign * jnp.uint32(0x7FFFFFFF)
    key = u ^ flip

    kk = jnp.int32(_K)

    tstar = jnp.uint32(0)
    rounds = [(28 - 4 * r, 15) for r in range(8)]
    for shift, nc in rounds:
        cnts = [
            jnp.sum((key >= (tstar + jnp.uint32(i << shift)))
                    .astype(jnp.int32))
            for i in range(1, nc + 1)
        ]
        m = jnp.uint32(0)
        for c in cnts:
            m = m + (c >= kk).astype(jnp.uint32)
        tstar = tstar + (m << jnp.uint32(shift))

    gt = key > tstar
    eq = key == tstar
    c = jnp.sum(gt.astype(jnp.int32))
    need = (kk - c).astype(jnp.float32)

    e = eq.astype(jnp.float32)
    i1 = jax.lax.broadcasted_iota(jnp.int32, (_C, _C), 0)
    j1 = jax.lax.broadcasted_iota(jnp.int32, (_C, _C), 1)
    upper = (i1 <= j1).astype(jnp.float32)
    rowcum = jnp.dot(e, upper, preferred_element_type=jnp.float32)

    i2 = jax.lax.broadcasted_iota(jnp.int32, (_R, _R), 0)
    j2 = jax.lax.broadcasted_iota(jnp.int32, (_R, _R), 1)
    strict_lower = (j2 < i2).astype(jnp.float32)
    colpref = jnp.dot(strict_lower, e, preferred_element_type=jnp.float32)
    row_prefix = jnp.sum(colpref, axis=1, keepdims=True)

    rank = rowcum + row_prefix
    sel = jnp.logical_and(eq, rank <= need)
    o_ref[...] = jnp.logical_or(gt, sel).astype(jnp.float32)


def kernel(x, gate_scores):
    g2 = gate_scores.reshape(_R, _C)
    mask = pl.pallas_call(
        _body,
        out_shape=jax.ShapeDtypeStruct((_R, _C), jnp.float32),
    )(g2)
    return mask.reshape(_N).astype(x.dtype)
